# baseline (speedup 1.0000x reference)
import functools

import jax
import jax.numpy as jnp
from jax import lax
from jax.experimental import pallas as pl
from jax.experimental.pallas import tpu as pltpu
from jax.experimental.pallas import tpu_sc as plsc

_L0 = 1000
_GROUP = 18432
_BLOCK = 2304
_MAXN = 512
_NGROUP = 8
_LANES = 16
_NROWS = 2 * _NGROUP


def _body(in_hbm, *refs):
    out0 = refs[0]
    outs = refs[1:1 + _NGROUP]
    rows_v = refs[1 + _NGROUP]
    l0_v = refs[2 + _NGROUP]
    sem_in = refs[3 + _NGROUP]
    sem_out = refs[4 + _NGROUP]
    sem_l0 = refs[5 + _NGROUP]

    wid = lax.axis_index("s") * 2 + lax.axis_index("c")

    neg_inf = jnp.full((_LANES,), -jnp.inf, dtype=jnp.float32)

    m0 = lax.rem(wid, 8)
    lr1 = 63 - wid
    off0 = _L0 + lax.div(wid, 8) * _BLOCK + 32 * m0 * (m0 + 1)
    m1 = 7 - m0
    off1 = _L0 + lax.div(lr1, 8) * _BLOCK + 32 * m1 * (m1 + 1)
    for k in range(8):
        @pl.when(m0 == k)
        def _(k=k):
            n0 = 64 * (k + 1)
            n1 = 64 * (8 - k)

            def _fire(g, _):
                pltpu.async_copy(
                    in_hbm.at[pl.ds(off0 + g * _GROUP, n0)],
                    rows_v.at[pl.ds(2 * g * _MAXN, n0)], sem_in)
                pltpu.async_copy(
                    in_hbm.at[pl.ds(off1 + g * _GROUP, n1)],
                    rows_v.at[pl.ds((2 * g + 1) * _MAXN, n1)], sem_in)
                return 0

            lax.fori_loop(0, _NGROUP, _fire, 0)

    @pl.when(wid == 0)
    def _():
        pltpu.async_copy(in_hbm.at[pl.ds(0, _L0)], l0_v, sem_l0)

    def _pad_row(i, _):
        t = lax.rem(i, 2)
        lr = wid + t * (63 - 2 * wid)
        m = lax.rem(lr, 8)
        base = i * _MAXN

        def _pad64(c, _):
            for k in range(4):
                rows_v[pl.ds(base + c * 64 + k * _LANES, _LANES)] = neg_inf
            return 0

        lax.fori_loop(m + 1, 8, _pad64, 0)
        return 0

    lax.fori_loop(0, _NROWS, _pad_row, 0)

    pltpu.make_async_copy(in_hbm.at[pl.ds(0, _NGROUP * 576)],
                          rows_v.at[pl.ds(0, _NGROUP * 576)], sem_in).wait()

    @pl.when(wid == 0)
    def _():
        pltpu.make_async_copy(in_hbm.at[pl.ds(0, _L0)], l0_v, sem_l0).wait()
        pltpu.async_copy(l0_v, out0.at[0], sem_l0)

    for g in range(_NGROUP):
        for t in range(2):
            lr = wid + t * (63 - 2 * wid)
            pltpu.async_copy(rows_v.at[pl.ds((2 * g + t) * _MAXN, _MAXN)],
                             outs[g].at[lr], sem_out)

    pltpu.make_async_copy(in_hbm.at[pl.ds(0, _NROWS * _MAXN)],
                          rows_v, sem_out).wait()

    @pl.when(wid == 0)
    def _():
        pltpu.make_async_copy(in_hbm.at[pl.ds(0, _L0)], l0_v, sem_l0).wait()


_OUT_TYPE = (
    (jax.ShapeDtypeStruct((1, _L0), jnp.float32),)
    + tuple(jax.ShapeDtypeStruct((64, _MAXN), jnp.float32)
            for _ in range(_NGROUP))
)

_sc_interpret = functools.partial(
    pl.kernel,
    mesh=plsc.VectorSubcoreMesh(core_axis_name="c", subcore_axis_name="s"),
    out_type=_OUT_TYPE,
    scratch_types=[
        pltpu.VMEM((_NROWS * _MAXN,), jnp.float32),
        pltpu.VMEM((_L0,), jnp.float32),
        pltpu.SemaphoreType.DMA,
        pltpu.SemaphoreType.DMA,
        pltpu.SemaphoreType.DMA,
    ],
)(_body)


def kernel(logits):
    return _sc_interpret(logits)


# speedup vs baseline: 1.0130x; 1.0130x over previous
import functools

import jax
import jax.numpy as jnp
from jax import lax
from jax.experimental import pallas as pl
from jax.experimental.pallas import tpu as pltpu
from jax.experimental.pallas import tpu_sc as plsc

_L0 = 1000
_GROUP = 18432
_BLOCK = 2304
_MAXN = 512
_NGROUP = 8
_LANES = 16
_NROWS = 2 * _NGROUP


def _body(in_hbm, *refs):
    out0 = refs[0]
    outs = refs[1:1 + _NGROUP]
    rows_v = refs[1 + _NGROUP]
    l0_v = refs[2 + _NGROUP]
    sem_in = refs[3 + _NGROUP]
    sem_out = refs[4 + _NGROUP]
    sem_l0 = refs[5 + _NGROUP]
    sem_in2 = refs[6 + _NGROUP]

    wid = lax.axis_index("s") * 2 + lax.axis_index("c")

    neg_inf = jnp.full((_LANES,), -jnp.inf, dtype=jnp.float32)

    def _fire(sem):
        def fire_i(i, _):
            t = lax.rem(i, 2)
            g = lax.div(i, 2)
            lr = wid + t * (63 - 2 * wid)
            m = lax.rem(lr, 8)
            blk = lax.div(lr, 8)
            in_off = _L0 + g * _GROUP + blk * _BLOCK + 32 * m * (m + 1)
            for k in range(8):
                @pl.when(m == k)
                def _(k=k):
                    nn = 64 * (k + 1)
                    pltpu.async_copy(in_hbm.at[pl.ds(in_off, nn)],
                                     rows_v.at[pl.ds(i * _MAXN, nn)], sem)
            return 0
        return fire_i

    lax.fori_loop(0, _NROWS // 2, _fire(sem_in), 0)
    lax.fori_loop(_NROWS // 2, _NROWS, _fire(sem_in2), 0)

    @pl.when(wid == 0)
    def _():
        pltpu.async_copy(in_hbm.at[pl.ds(0, _L0)], l0_v, sem_l0)

    def _pad_row(i, _):
        t = lax.rem(i, 2)
        lr = wid + t * (63 - 2 * wid)
        m = lax.rem(lr, 8)
        base = i * _MAXN

        def _pad64(c, _):
            for k in range(4):
                rows_v[pl.ds(base + c * 64 + k * _LANES, _LANES)] = neg_inf
            return 0

        lax.fori_loop(m + 1, 8, _pad64, 0)
        return 0

    lax.fori_loop(0, _NROWS, _pad_row, 0)

    def _scatter(g, t):
        lr = wid + t * (63 - 2 * wid)
        pltpu.async_copy(rows_v.at[pl.ds((2 * g + t) * _MAXN, _MAXN)],
                         outs[g].at[lr], sem_out)

    pltpu.make_async_copy(in_hbm.at[pl.ds(0, _NGROUP // 2 * 576)],
                          rows_v.at[pl.ds(0, _NGROUP // 2 * 576)],
                          sem_in).wait()

    @pl.when(wid == 0)
    def _():
        pltpu.make_async_copy(in_hbm.at[pl.ds(0, _L0)], l0_v, sem_l0).wait()
        pltpu.async_copy(l0_v, out0.at[0], sem_l0)

    for g in range(_NGROUP // 2):
        for t in range(2):
            _scatter(g, t)

    pltpu.make_async_copy(in_hbm.at[pl.ds(0, _NGROUP // 2 * 576)],
                          rows_v.at[pl.ds(0, _NGROUP // 2 * 576)],
                          sem_in2).wait()
    for g in range(_NGROUP // 2, _NGROUP):
        for t in range(2):
            _scatter(g, t)

    pltpu.make_async_copy(in_hbm.at[pl.ds(0, _NROWS * _MAXN)],
                          rows_v, sem_out).wait()

    @pl.when(wid == 0)
    def _():
        pltpu.make_async_copy(in_hbm.at[pl.ds(0, _L0)], l0_v, sem_l0).wait()


_OUT_TYPE = (
    (jax.ShapeDtypeStruct((1, _L0), jnp.float32),)
    + tuple(jax.ShapeDtypeStruct((64, _MAXN), jnp.float32)
            for _ in range(_NGROUP))
)

_sc_interpret = functools.partial(
    pl.kernel,
    mesh=plsc.VectorSubcoreMesh(core_axis_name="c", subcore_axis_name="s"),
    out_type=_OUT_TYPE,
    scratch_types=[
        pltpu.VMEM((_NROWS * _MAXN,), jnp.float32),
        pltpu.VMEM((_L0,), jnp.float32),
        pltpu.SemaphoreType.DMA,
        pltpu.SemaphoreType.DMA,
        pltpu.SemaphoreType.DMA,
        pltpu.SemaphoreType.DMA,
    ],
)(_body)


def kernel(logits):
    return _sc_interpret(logits)
